# Initial kernel scaffold; baseline (speedup 1.0000x reference)
import jax
import jax.numpy as jnp
from jax.experimental import pallas as pl


def kernel(x, edge_index, W1_l, b1_l, W1_r, W2_l, b2_l, W2_r):
    raise NotImplementedError("write your pallas kernel here")


# speedup vs baseline: 3.6535x; 3.6535x over previous
import functools

import jax
import jax.numpy as jnp
from jax import lax
from jax.experimental import pallas as pl
from jax.experimental.pallas import tpu as pltpu
from jax.experimental.pallas import tpu_sc as plsc

N_NODES = 10000
D_IN = 128
D_OUT = 64

NC = 2
NS = 16
NW = NC * NS
K = 128
E = 320000
E_PAD = 327680
CPW = E_PAD // (NW * K)
N_PAD = 10240
ROWS_PER_TILE = N_PAD // NS


def _make_agg(D, with_deg):
  mesh = plsc.VectorSubcoreMesh(core_axis_name="c", subcore_axis_name="s")
  outs = [jax.ShapeDtypeStruct((NC, N_PAD, D), jnp.float32)]
  scratch = [
      pltpu.VMEM((K,), jnp.int32),
      pltpu.VMEM((K,), jnp.int32),
      pltpu.VMEM((K, D), jnp.float32),
      pltpu.VMEM_SHARED((N_PAD, D), jnp.float32),
      pltpu.SemaphoreType.DMA,
  ]
  if with_deg:
    outs.append(jax.ShapeDtypeStruct((NC, N_PAD), jnp.float32))
    scratch += [
        pltpu.VMEM((K,), jnp.float32),
        pltpu.VMEM_SHARED((N_PAD,), jnp.float32),
    ]

  params = {}
  if D % 128 != 0:
    params["compiler_params"] = pltpu.CompilerParams(
        use_tc_tiling_on_sc=False)

  @functools.partial(
      pl.kernel, mesh=mesh, out_type=outs, scratch_types=scratch, **params)
  def agg(table, src2, dst2, znd, zn, *rest):
    if with_deg:
      out, deg_out, sidx, didx, rows, acc, sem, ones, dacc = rest
    else:
      out, sidx, didx, rows, acc, sem = rest
    c = lax.axis_index("c")
    s = lax.axis_index("s")
    wid = c * NS + s
    r0 = s * ROWS_PER_TILE
    pltpu.sync_copy(znd.at[pl.ds(r0, ROWS_PER_TILE)],
                    acc.at[pl.ds(r0, ROWS_PER_TILE)])
    if with_deg:
      pltpu.sync_copy(zn.at[pl.ds(r0, ROWS_PER_TILE)],
                      dacc.at[pl.ds(r0, ROWS_PER_TILE)])
      for j in range(K // 16):
        ones[pl.ds(16 * j, 16)] = jnp.ones((16,), jnp.float32)
    plsc.subcore_barrier()

    def chunk(i, carry):
      r = wid * CPW + i
      pltpu.sync_copy(src2.at[r], sidx)
      pltpu.sync_copy(dst2.at[r], didx)
      pltpu.async_copy(table.at[sidx], rows, sem).wait()
      pltpu.sync_copy(rows, acc.at[didx], add=True)
      if with_deg:
        pltpu.sync_copy(ones, dacc.at[didx], add=True)
      return carry

    lax.fori_loop(0, CPW, chunk, 0)

    plsc.subcore_barrier()
    pltpu.sync_copy(acc.at[pl.ds(r0, ROWS_PER_TILE)],
                    out.at[c, pl.ds(r0, ROWS_PER_TILE)])
    if with_deg:
      pltpu.sync_copy(dacc.at[pl.ds(r0, ROWS_PER_TILE)],
                      deg_out.at[c, pl.ds(r0, ROWS_PER_TILE)])

  return agg


_agg128 = _make_agg(D_IN, with_deg=True)
_agg64 = _make_agg(D_OUT, with_deg=False)


def _mid_body(p0, p1, d0, d1, x, w1l, b1, w1r, w2l, w2r, g_out, r2_out,
              inv_out):
  inv = 1.0 / jnp.maximum(d0[...] + d1[...], 1.0)
  mean = (p0[...] + p1[...]) * inv
  h = jnp.maximum(
      jnp.dot(mean, w1l[...], preferred_element_type=jnp.float32) + b1[...] +
      jnp.dot(x[...], w1r[...], preferred_element_type=jnp.float32), 0.0)
  g_out[...] = jnp.dot(h, w2l[...], preferred_element_type=jnp.float32)
  r2_out[...] = jnp.dot(h, w2r[...], preferred_element_type=jnp.float32)
  inv_out[...] = inv


_mid = pl.pallas_call(
    _mid_body,
    out_shape=[
        jax.ShapeDtypeStruct((N_NODES, D_OUT), jnp.float32),
        jax.ShapeDtypeStruct((N_NODES, D_OUT), jnp.float32),
        jax.ShapeDtypeStruct((N_NODES, 1), jnp.float32),
    ],
)


def _out_body(q0, q1, inv, r2, b2, o):
  z = (q0[...] + q1[...]) * inv[...] + b2[...] + r2[...]
  m = jnp.max(z, axis=1, keepdims=True)
  lse = jnp.log(jnp.sum(jnp.exp(z - m), axis=1, keepdims=True)) + m
  o[...] = z - lse


_outk = pl.pallas_call(
    _out_body,
    out_shape=jax.ShapeDtypeStruct((N_NODES, D_OUT), jnp.float32),
)


def kernel(x, edge_index, W1_l, b1_l, W1_r, W2_l, b2_l, W2_r):
  src = edge_index[0].astype(jnp.int32)
  dst = edge_index[1].astype(jnp.int32)
  pad = E_PAD - E
  src2 = jnp.concatenate([src, jnp.zeros((pad,), jnp.int32)]).reshape(-1, K)
  dst2 = jnp.concatenate(
      [dst, jnp.full((pad,), N_PAD - 1, jnp.int32)]).reshape(-1, K)
  znd = jnp.zeros((N_PAD, D_IN), jnp.float32)
  znd64 = jnp.zeros((N_PAD, D_OUT), jnp.float32)
  zn = jnp.zeros((N_PAD,), jnp.float32)

  p, degp = _agg128(x, src2, dst2, znd, zn)
  g, r2, inv = _mid(p[0, :N_NODES], p[1, :N_NODES],
                    degp[0, :N_NODES, None], degp[1, :N_NODES, None],
                    x, W1_l, b1_l.reshape(1, -1), W1_r, W2_l, W2_r)
  q, = _agg64(g, src2, dst2, znd64, zn)
  out = _outk(q[0, :N_NODES], q[1, :N_NODES], inv, r2, b2_l.reshape(1, -1))
  return out
